# Initial kernel scaffold; baseline (speedup 1.0000x reference)
import jax
import jax.numpy as jnp
from jax.experimental import pallas as pl


def kernel(x, pos, batch, W1, b1, W2, b2, W3, b3):
    raise NotImplementedError("write your pallas kernel here")


# speedup vs baseline: 1.1437x; 1.1437x over previous
import functools

import jax
import jax.numpy as jnp
from jax import lax
from jax.experimental import pallas as pl

B = 8
P = 1024
S = 512
K = 64
R2 = 0.04
F = 8


def _fps_body(px_ref, py_ref, pz_ref, ox_ref, oy_ref, oz_ref):
    px = px_ref[...]
    py = py_ref[...]
    pz = pz_ref[...]
    lane = lax.broadcasted_iota(jnp.int32, (B, P), 1)
    lane_s = lax.broadcasted_iota(jnp.int32, (B, S), 1)

    def step(i, carry):
        dist, far, ax, ay, az = carry
        onehot = lane == far
        cx = jnp.sum(jnp.where(onehot, px, 0.0), axis=1, keepdims=True)
        cy = jnp.sum(jnp.where(onehot, py, 0.0), axis=1, keepdims=True)
        cz = jnp.sum(jnp.where(onehot, pz, 0.0), axis=1, keepdims=True)
        slot = lane_s == i
        ax = jnp.where(slot, cx, ax)
        ay = jnp.where(slot, cy, ay)
        az = jnp.where(slot, cz, az)
        d = (px - cx) ** 2 + (py - cy) ** 2 + (pz - cz) ** 2
        dist = jnp.minimum(dist, d)
        m = jnp.max(dist, axis=1, keepdims=True)
        cand = jnp.where(dist == m, lane, jnp.int32(1 << 30))
        far = jnp.min(cand, axis=1, keepdims=True)
        return dist, far, ax, ay, az

    dist0 = jnp.full((B, P), jnp.inf, dtype=jnp.float32)
    far0 = jnp.zeros((B, 1), dtype=jnp.int32)
    z = jnp.zeros((B, S), dtype=jnp.float32)
    _, _, ax, ay, az = lax.fori_loop(0, S, step, (dist0, far0, z, z, z))
    ox_ref[...] = ax
    oy_ref[...] = ay
    oz_ref[...] = az


def _run_fps(px, py, pz):
    out = jax.ShapeDtypeStruct((B, S), jnp.float32)
    return pl.pallas_call(
        _fps_body,
        out_shape=[out, out, out],
    )(px, py, pz)


def _mlp_body(h_ref, w1_ref, b1_ref, w2_ref, b2_ref, w3_ref, b3_ref, o_ref,
              *, rows):
    h = h_ref[...]
    g = jnp.maximum(jnp.dot(h, w1_ref[...],
                            preferred_element_type=jnp.float32) + b1_ref[...], 0.0)
    g = jnp.maximum(jnp.dot(g, w2_ref[...],
                            preferred_element_type=jnp.float32) + b2_ref[...], 0.0)
    g = jnp.dot(g, w3_ref[...], preferred_element_type=jnp.float32) + b3_ref[...]
    o_ref[...] = jnp.max(g.reshape(rows // K, K, 128), axis=1)


def _run_mlp(h2d, W1p, b1, W2, b2, W3, b3):
    rows_per_blk = 4096
    n_blk = (B * S * K) // rows_per_blk
    full = lambda shape: pl.BlockSpec(shape, lambda i: (0, 0))
    return pl.pallas_call(
        functools.partial(_mlp_body, rows=rows_per_blk),
        grid=(n_blk,),
        in_specs=[
            pl.BlockSpec((rows_per_blk, F), lambda i: (i, 0)),
            full((F, 64)), full((1, 64)),
            full((64, 64)), full((1, 64)),
            full((64, 128)), full((1, 128)),
        ],
        out_specs=pl.BlockSpec((rows_per_blk // K, 128), lambda i: (i, 0)),
        out_shape=jax.ShapeDtypeStruct((B * S, 128), jnp.float32),
    )(h2d, W1p, b1, W2, b2, W3, b3)


def _ball_gather(x, pos, psx, psy, psz):
    pos_b = pos.reshape(B, P, 3)
    x_b = x.reshape(B, P, 3)
    pos_s = jnp.stack([psx, psy, psz], axis=-1)
    d2 = jnp.sum((pos_s[:, :, None, :] - pos_b[:, None, :, :]) ** 2, axis=-1)
    d2m = jnp.where(d2 <= R2, d2, jnp.inf)
    neg, nbr = lax.top_k(-d2m, K)
    valid = neg > -jnp.inf
    nbr = jnp.where(valid, nbr, nbr[:, :, 0:1])
    x_j = jax.vmap(lambda xb, nb: xb[nb])(x_b, nbr)
    pos_j = jax.vmap(lambda pb, nb: pb[nb])(pos_b, nbr)
    rel = pos_j - pos_s[:, :, None, :]
    h = jnp.concatenate([x_j, rel], axis=-1)
    h = jnp.pad(h, ((0, 0), (0, 0), (0, 0), (0, 2)))
    return h.reshape(B * S * K, F), pos_s


def kernel(x, pos, batch, W1, b1, W2, b2, W3, b3):
    pos_b = pos.reshape(B, P, 3)
    px = pos_b[:, :, 0]
    py = pos_b[:, :, 1]
    pz = pos_b[:, :, 2]
    psx, psy, psz = _run_fps(px, py, pz)

    h2d, pos_s = _ball_gather(x, pos, psx, psy, psz)

    W1p = jnp.pad(W1, ((0, 2), (0, 0)))
    out_x = _run_mlp(h2d, W1p, b1.reshape(1, 64), W2, b2.reshape(1, 64),
                     W3, b3.reshape(1, 128))
    out_pos = pos_s.reshape(B * S, 3)
    out_batch = jnp.repeat(jnp.arange(B, dtype=jnp.int32), S)
    return (out_x, out_pos, out_batch)
